# baseline (speedup 1.0000x reference)
import functools
import math

import jax
import jax.numpy as jnp
from jax import lax
from jax.experimental import pallas as pl
from jax.experimental.pallas import tpu as pltpu
from jax.experimental.pallas import tpu_sc as plsc

N_REF = 8192
N_PRED = 8192
C = 512
K = 8
BK = 512
NBK = N_PRED // BK
B = 256
NBQ = N_PRED // B
SCALE = 1.0 / math.sqrt(C)

SC_NC = 2
SC_NS = 16
SC_NW = SC_NC * SC_NS
GCH = 64


def _knn_body(xp_ref, xrt_ref, idx_ref):
    xp = xp_ref[...]
    xpb = xp.astype(jnp.bfloat16).astype(jnp.float32)
    x0 = xrt_ref[0:1, :]
    x1 = xrt_ref[1:2, :]
    x2 = xrt_ref[2:3, :]
    x0b = x0.astype(jnp.bfloat16).astype(jnp.float32)
    x1b = x1.astype(jnp.bfloat16).astype(jnp.float32)
    x2b = x2.astype(jnp.bfloat16).astype(jnp.float32)
    xp2 = jnp.sum(xp * xp, axis=1, keepdims=True)
    xr2 = x0 * x0 + x1 * x1 + x2 * x2
    dot = (xpb[:, 0:1] * x0b + xpb[:, 1:2] * x1b
           + xpb[:, 2:3] * x2b)
    d = xp2 - 2.0 * dot + xr2
    col = lax.broadcasted_iota(jnp.int32, (BK, N_REF), 1)
    picks = []
    for _ in range(K):
        m = jnp.min(d, axis=1, keepdims=True)
        t = jnp.where(d == m, col, N_REF)
        sel = jnp.min(t, axis=1, keepdims=True)
        picks.append(sel)
        d = jnp.where(t == sel, jnp.inf, d)
    idx_ref[...] = jnp.concatenate(picks, axis=1)


def _combine_body(wv_ref, wo_ref, wout_ref, bv_ref, bo_ref, bout_ref,
                  m_ref, c_ref):
    a = jnp.dot(wo_ref[...], wv_ref[...],
                preferred_element_type=jnp.float32)
    m_ref[...] = lax.dot_general(a, wout_ref[...], (((0,), (0,)), ((), ())),
                                 preferred_element_type=jnp.float32)
    cb = lax.dot_general(bv_ref[...], wo_ref[...], (((1,), (1,)), ((), ())),
                         preferred_element_type=jnp.float32) + bo_ref[...]
    c_ref[...] = jnp.dot(cb, wout_ref[...],
                         preferred_element_type=jnp.float32) + bout_ref[...]


def _attn_body(q_ref, kg_ref, sg_ref, m_ref, c_ref, bout_ref, out_ref):
    pid = pl.program_id(0)

    @pl.when(pid < NBQ)
    def _():
        out_ref[...] = jnp.broadcast_to(bout_ref[...], (B, 2 * C))

    @pl.when(pid >= NBQ)
    def _():
        q = q_ref[...]
        qb = q.astype(jnp.bfloat16).astype(jnp.float32)
        kg = kg_ref[...].astype(jnp.float32).reshape(B, K, C)
        sg = sg_ref[...].astype(jnp.float32).reshape(B, K, C)
        s = jnp.sum(qb[:, None, :] * kg, axis=2) * SCALE
        mx = jnp.max(s, axis=1, keepdims=True)
        e = jnp.exp(s - mx)
        a = e / jnp.sum(e, axis=1, keepdims=True)
        osum = jnp.sum(a[:, :, None] * sg, axis=1)
        out_ref[...] = jnp.dot(osum, m_ref[...],
                               preferred_element_type=jnp.float32) + c_ref[...]


def _sc_gather(fc, fs, idxf):
    n_idx = idxf.shape[0]
    b_per_w = n_idx // SC_NW
    n_pair = b_per_w // (2 * GCH)
    c2 = C // 2
    mesh = plsc.VectorSubcoreMesh(core_axis_name="c", subcore_axis_name="s")

    @functools.partial(
        pl.kernel, mesh=mesh,
        out_type=[jax.ShapeDtypeStruct((n_idx, c2), jnp.int32),
                  jax.ShapeDtypeStruct((n_idx, c2), jnp.int32)],
        scratch_types=[pltpu.VMEM((GCH,), jnp.int32),
                       pltpu.VMEM((GCH,), jnp.int32),
                       pltpu.VMEM((GCH, c2), jnp.int32),
                       pltpu.VMEM((GCH, c2), jnp.int32),
                       pltpu.VMEM((GCH, c2), jnp.int32),
                       pltpu.VMEM((GCH, c2), jnp.int32),
                       pltpu.SemaphoreType.DMA,
                       pltpu.SemaphoreType.DMA,
                       pltpu.SemaphoreType.DMA,
                       pltpu.SemaphoreType.DMA],
    )
    def gather_k(fc_hbm, fs_hbm, idx_hbm, kg_hbm, sg_hbm,
                 idx0, idx1, a0, b0, a1, b1, sa0, sb0, sa1, sb1):
        wid = lax.axis_index("s") * SC_NC + lax.axis_index("c")

        def body(t, carry):
            base0 = wid * b_per_w + 2 * t * GCH
            base1 = base0 + GCH
            pltpu.sync_copy(idx_hbm.at[pl.ds(base0, GCH)], idx0)
            ca0 = pltpu.async_copy(fc_hbm.at[idx0], a0, sa0)
            cb0 = pltpu.async_copy(fs_hbm.at[idx0], b0, sb0)
            pltpu.sync_copy(idx_hbm.at[pl.ds(base1, GCH)], idx1)
            ca1 = pltpu.async_copy(fc_hbm.at[idx1], a1, sa1)
            cb1 = pltpu.async_copy(fs_hbm.at[idx1], b1, sb1)
            ca0.wait()
            pltpu.sync_copy(a0, kg_hbm.at[pl.ds(base0, GCH)])
            cb0.wait()
            pltpu.sync_copy(b0, sg_hbm.at[pl.ds(base0, GCH)])
            ca1.wait()
            pltpu.sync_copy(a1, kg_hbm.at[pl.ds(base1, GCH)])
            cb1.wait()
            pltpu.sync_copy(b1, sg_hbm.at[pl.ds(base1, GCH)])
            return carry

        lax.fori_loop(0, n_pair, body, 0)

    return gather_k(fc, fs, idxf)


def kernel(xyz_ref, xyz_pred, feat_coor_ref, feat_coor_pred, feat_sp_ref,
           W_v, b_v, W_o, b_o, W_out, b_out):
    idx = pl.pallas_call(
        _knn_body,
        grid=(NBK,),
        in_specs=[pl.BlockSpec((BK, 3), lambda i: (i, 0)),
                  pl.BlockSpec((3, N_REF), lambda i: (0, 0))],
        out_specs=pl.BlockSpec((BK, K), lambda i: (i, 0)),
        out_shape=jax.ShapeDtypeStruct((N_PRED, K), jnp.int32),
    )(xyz_pred, xyz_ref.T)

    def _to_i32(x):
        xb = x.astype(jnp.bfloat16).reshape(x.shape[0], x.shape[1] // 2, 2)
        return lax.bitcast_convert_type(xb, jnp.int32)

    def _to_bf16(x32):
        return lax.bitcast_convert_type(
            x32, jnp.bfloat16).reshape(x32.shape[0], 2 * x32.shape[1])

    kg32, sg32 = _sc_gather(_to_i32(feat_coor_ref), _to_i32(feat_sp_ref),
                            idx.reshape(-1))
    kg, sg = _to_bf16(kg32), _to_bf16(sg32)

    m, c = pl.pallas_call(
        _combine_body,
        out_shape=(jax.ShapeDtypeStruct((C, 2 * C), jnp.float32),
                   jax.ShapeDtypeStruct((1, 2 * C), jnp.float32)),
    )(W_v, W_o, W_out, b_v.reshape(1, C), b_o.reshape(1, C),
      b_out.reshape(1, 2 * C))

    out = pl.pallas_call(
        _attn_body,
        grid=(2 * NBQ,),
        in_specs=[
            pl.BlockSpec((B, C), lambda i: (jnp.maximum(i - NBQ, 0), 0)),
            pl.BlockSpec((B * K, C), lambda i: (jnp.maximum(i - NBQ, 0), 0)),
            pl.BlockSpec((B * K, C), lambda i: (jnp.maximum(i - NBQ, 0), 0)),
            pl.BlockSpec((C, 2 * C), lambda i: (0, 0)),
            pl.BlockSpec((1, 2 * C), lambda i: (0, 0)),
            pl.BlockSpec((1, 2 * C), lambda i: (0, 0)),
        ],
        out_specs=pl.BlockSpec((B, 2 * C), lambda i: (i, 0)),
        out_shape=jax.ShapeDtypeStruct((N_REF + N_PRED, 2 * C), jnp.float32),
    )(feat_coor_pred, kg, sg, m, c, b_out.reshape(1, 2 * C))
    return out


# speedup vs baseline: 1.0666x; 1.0666x over previous
import functools
import math

import jax
import jax.numpy as jnp
from jax import lax
from jax.experimental import pallas as pl
from jax.experimental.pallas import tpu as pltpu
from jax.experimental.pallas import tpu_sc as plsc

N_REF = 8192
N_PRED = 8192
C = 512
K = 8
BK = 256
NBK = N_PRED // BK
B = 256
NBQ = N_PRED // B
SCALE = 1.0 / math.sqrt(C)

SC_NC = 2
SC_NS = 16
SC_NW = SC_NC * SC_NS
GCH = 64


def _knn_body(xp_ref, xrt_ref, idx_ref):
    xp = xp_ref[...]
    xpb = xp.astype(jnp.bfloat16).astype(jnp.float32)
    x0 = xrt_ref[0:1, :]
    x1 = xrt_ref[1:2, :]
    x2 = xrt_ref[2:3, :]
    x0b = x0.astype(jnp.bfloat16).astype(jnp.float32)
    x1b = x1.astype(jnp.bfloat16).astype(jnp.float32)
    x2b = x2.astype(jnp.bfloat16).astype(jnp.float32)
    xp2 = jnp.sum(xp * xp, axis=1, keepdims=True)
    xr2 = x0 * x0 + x1 * x1 + x2 * x2
    dot = (xpb[:, 0:1] * x0b + xpb[:, 1:2] * x1b
           + xpb[:, 2:3] * x2b)
    d = xp2 - 2.0 * dot + xr2
    col = lax.broadcasted_iota(jnp.int32, (BK, N_REF), 1)
    picks = []
    for _ in range(K):
        m = jnp.min(d, axis=1, keepdims=True)
        t = jnp.where(d == m, col, N_REF)
        sel = jnp.min(t, axis=1, keepdims=True)
        picks.append(sel)
        d = jnp.where(t == sel, jnp.inf, d)
    idx_ref[...] = jnp.concatenate(picks, axis=1)


def _combine_body(wv_ref, wo_ref, wout_ref, bv_ref, bo_ref, bout_ref,
                  m_ref, c_ref):
    a = jnp.dot(wo_ref[...], wv_ref[...],
                preferred_element_type=jnp.float32)
    m_ref[...] = lax.dot_general(a, wout_ref[...], (((0,), (0,)), ((), ())),
                                 preferred_element_type=jnp.float32)
    cb = lax.dot_general(bv_ref[...], wo_ref[...], (((1,), (1,)), ((), ())),
                         preferred_element_type=jnp.float32) + bo_ref[...]
    c_ref[...] = jnp.dot(cb, wout_ref[...],
                         preferred_element_type=jnp.float32) + bout_ref[...]


def _attn_body(q_ref, kg_ref, sg_ref, m_ref, c_ref, bout_ref, out_ref):
    pid = pl.program_id(0)

    @pl.when(pid < NBQ)
    def _():
        out_ref[...] = jnp.broadcast_to(bout_ref[...], (B, 2 * C))

    @pl.when(pid >= NBQ)
    def _():
        q = q_ref[...]
        qb = q.astype(jnp.bfloat16).astype(jnp.float32)
        kg = kg_ref[...].astype(jnp.float32).reshape(B, K, C)
        sg = sg_ref[...].astype(jnp.float32).reshape(B, K, C)
        s = jnp.sum(qb[:, None, :] * kg, axis=2) * SCALE
        mx = jnp.max(s, axis=1, keepdims=True)
        e = jnp.exp(s - mx)
        a = e / jnp.sum(e, axis=1, keepdims=True)
        osum = jnp.sum(a[:, :, None] * sg, axis=1)
        out_ref[...] = jnp.dot(osum, m_ref[...],
                               preferred_element_type=jnp.float32) + c_ref[...]


def _sc_gather(fc, fs, idxf):
    n_idx = idxf.shape[0]
    b_per_w = n_idx // SC_NW
    n_pair = b_per_w // (2 * GCH)
    c2 = C // 2
    mesh = plsc.VectorSubcoreMesh(core_axis_name="c", subcore_axis_name="s")

    @functools.partial(
        pl.kernel, mesh=mesh,
        out_type=[jax.ShapeDtypeStruct((n_idx, c2), jnp.int32),
                  jax.ShapeDtypeStruct((n_idx, c2), jnp.int32)],
        scratch_types=[pltpu.VMEM((GCH,), jnp.int32),
                       pltpu.VMEM((GCH,), jnp.int32),
                       pltpu.VMEM((GCH, c2), jnp.int32),
                       pltpu.VMEM((GCH, c2), jnp.int32),
                       pltpu.VMEM((GCH, c2), jnp.int32),
                       pltpu.VMEM((GCH, c2), jnp.int32),
                       pltpu.SemaphoreType.DMA,
                       pltpu.SemaphoreType.DMA,
                       pltpu.SemaphoreType.DMA,
                       pltpu.SemaphoreType.DMA],
    )
    def gather_k(fc_hbm, fs_hbm, idx_hbm, kg_hbm, sg_hbm,
                 idx0, idx1, a0, b0, a1, b1, sa0, sb0, sa1, sb1):
        wid = lax.axis_index("s") * SC_NC + lax.axis_index("c")

        def body(t, carry):
            base0 = wid * b_per_w + 2 * t * GCH
            base1 = base0 + GCH
            pltpu.sync_copy(idx_hbm.at[pl.ds(base0, GCH)], idx0)
            ca0 = pltpu.async_copy(fc_hbm.at[idx0], a0, sa0)
            cb0 = pltpu.async_copy(fs_hbm.at[idx0], b0, sb0)
            pltpu.sync_copy(idx_hbm.at[pl.ds(base1, GCH)], idx1)
            ca1 = pltpu.async_copy(fc_hbm.at[idx1], a1, sa1)
            cb1 = pltpu.async_copy(fs_hbm.at[idx1], b1, sb1)
            ca0.wait()
            pltpu.sync_copy(a0, kg_hbm.at[pl.ds(base0, GCH)])
            cb0.wait()
            pltpu.sync_copy(b0, sg_hbm.at[pl.ds(base0, GCH)])
            ca1.wait()
            pltpu.sync_copy(a1, kg_hbm.at[pl.ds(base1, GCH)])
            cb1.wait()
            pltpu.sync_copy(b1, sg_hbm.at[pl.ds(base1, GCH)])
            return carry

        lax.fori_loop(0, n_pair, body, 0)

    return gather_k(fc, fs, idxf)


def kernel(xyz_ref, xyz_pred, feat_coor_ref, feat_coor_pred, feat_sp_ref,
           W_v, b_v, W_o, b_o, W_out, b_out):
    idx = pl.pallas_call(
        _knn_body,
        grid=(NBK,),
        in_specs=[pl.BlockSpec((BK, 3), lambda i: (i, 0)),
                  pl.BlockSpec((3, N_REF), lambda i: (0, 0))],
        out_specs=pl.BlockSpec((BK, K), lambda i: (i, 0)),
        out_shape=jax.ShapeDtypeStruct((N_PRED, K), jnp.int32),
    )(xyz_pred, xyz_ref.T)

    def _to_i32(x):
        xb = x.astype(jnp.bfloat16).reshape(x.shape[0], x.shape[1] // 2, 2)
        return lax.bitcast_convert_type(xb, jnp.int32)

    def _to_bf16(x32):
        return lax.bitcast_convert_type(
            x32, jnp.bfloat16).reshape(x32.shape[0], 2 * x32.shape[1])

    kg32, sg32 = _sc_gather(_to_i32(feat_coor_ref), _to_i32(feat_sp_ref),
                            idx.reshape(-1))
    kg, sg = _to_bf16(kg32), _to_bf16(sg32)

    m, c = pl.pallas_call(
        _combine_body,
        out_shape=(jax.ShapeDtypeStruct((C, 2 * C), jnp.float32),
                   jax.ShapeDtypeStruct((1, 2 * C), jnp.float32)),
    )(W_v, W_o, W_out, b_v.reshape(1, C), b_o.reshape(1, C),
      b_out.reshape(1, 2 * C))

    out = pl.pallas_call(
        _attn_body,
        grid=(2 * NBQ,),
        in_specs=[
            pl.BlockSpec((B, C), lambda i: (jnp.maximum(i - NBQ, 0), 0)),
            pl.BlockSpec((B * K, C), lambda i: (jnp.maximum(i - NBQ, 0), 0)),
            pl.BlockSpec((B * K, C), lambda i: (jnp.maximum(i - NBQ, 0), 0)),
            pl.BlockSpec((C, 2 * C), lambda i: (0, 0)),
            pl.BlockSpec((1, 2 * C), lambda i: (0, 0)),
            pl.BlockSpec((1, 2 * C), lambda i: (0, 0)),
        ],
        out_specs=pl.BlockSpec((B, 2 * C), lambda i: (i, 0)),
        out_shape=jax.ShapeDtypeStruct((N_REF + N_PRED, 2 * C), jnp.float32),
    )(feat_coor_pred, kg, sg, m, c, b_out.reshape(1, 2 * C))
    return out


# speedup vs baseline: 2.4269x; 2.2754x over previous
import functools
import math

import jax
import jax.numpy as jnp
from jax import lax
from jax.experimental import pallas as pl
from jax.experimental.pallas import tpu as pltpu
from jax.experimental.pallas import tpu_sc as plsc

N_REF = 8192
N_PRED = 8192
C = 512
K = 8
B = 256
NBQ = N_PRED // B
SCALE = 1.0 / math.sqrt(C)

SC_NC = 2
SC_NS = 16
SC_NW = SC_NC * SC_NS
GCH = 64


def _knn_body(xp_ref, xrt_ref, idx_ref):
    xp = xp_ref[...]
    xpb = xp.astype(jnp.bfloat16).astype(jnp.float32)
    x0 = xrt_ref[0:1, :]
    x1 = xrt_ref[1:2, :]
    x2 = xrt_ref[2:3, :]
    x0b = x0.astype(jnp.bfloat16).astype(jnp.float32)
    x1b = x1.astype(jnp.bfloat16).astype(jnp.float32)
    x2b = x2.astype(jnp.bfloat16).astype(jnp.float32)
    xp2 = jnp.sum(xp * xp, axis=1, keepdims=True)
    xr2 = x0 * x0 + x1 * x1 + x2 * x2
    dot = (xpb[:, 0:1] * x0b + xpb[:, 1:2] * x1b
           + xpb[:, 2:3] * x2b)
    d = xp2 - 2.0 * dot + xr2
    col = lax.broadcasted_iota(jnp.int32, (B, N_REF), 1)
    picks = []
    for _ in range(K):
        m = jnp.min(d, axis=1, keepdims=True)
        sel = jnp.min(jnp.where(d == m, col, N_REF), axis=1, keepdims=True)
        picks.append(sel)
        d = jnp.where(col == sel, jnp.inf, d)
    idx_ref[...] = jnp.concatenate(picks, axis=1)


def _combine_body(wv_ref, wo_ref, wout_ref, bv_ref, bo_ref, bout_ref,
                  m_ref, c_ref):
    a = jnp.dot(wo_ref[...], wv_ref[...],
                preferred_element_type=jnp.float32)
    m_ref[...] = lax.dot_general(a, wout_ref[...], (((0,), (0,)), ((), ())),
                                 preferred_element_type=jnp.float32)
    cb = lax.dot_general(bv_ref[...], wo_ref[...], (((1,), (1,)), ((), ())),
                         preferred_element_type=jnp.float32) + bo_ref[...]
    c_ref[...] = jnp.dot(cb, wout_ref[...],
                         preferred_element_type=jnp.float32) + bout_ref[...]


def _attn_body(q_ref, kg_ref, sg_ref, m_ref, c_ref, bout_ref, out_ref):
    pid = pl.program_id(0)

    @pl.when(pid < NBQ)
    def _():
        out_ref[...] = jnp.broadcast_to(bout_ref[...], (B, 2 * C))

    @pl.when(pid >= NBQ)
    def _():
        q = q_ref[...]
        kg = kg_ref[...].reshape(B, K, C)
        sg = sg_ref[...].reshape(B, K, C)
        s = jnp.sum(q[:, None, :] * kg, axis=2) * SCALE
        mx = jnp.max(s, axis=1, keepdims=True)
        e = jnp.exp(s - mx)
        a = e / jnp.sum(e, axis=1, keepdims=True)
        osum = jnp.sum(a[:, :, None] * sg, axis=1)
        out_ref[...] = jnp.dot(osum, m_ref[...],
                               preferred_element_type=jnp.float32) + c_ref[...]


def _sc_gather(fc, fs, idxf):
    n_idx = idxf.shape[0]
    b_per_w = n_idx // SC_NW
    n_ch = b_per_w // GCH
    mesh = plsc.VectorSubcoreMesh(core_axis_name="c", subcore_axis_name="s")

    @functools.partial(
        pl.kernel, mesh=mesh,
        out_type=[jax.ShapeDtypeStruct((n_idx, C), jnp.float32),
                  jax.ShapeDtypeStruct((n_idx, C), jnp.float32)],
        scratch_types=[pltpu.VMEM((GCH,), jnp.int32),
                       pltpu.VMEM((GCH, C), jnp.float32),
                       pltpu.VMEM((GCH, C), jnp.float32),
                       pltpu.SemaphoreType.DMA,
                       pltpu.SemaphoreType.DMA],
    )
    def gather_k(fc_hbm, fs_hbm, idx_hbm, kg_hbm, sg_hbm,
                 idx_v, rows_a, rows_b, sem_a, sem_b):
        wid = lax.axis_index("s") * SC_NC + lax.axis_index("c")

        def body(t, carry):
            base = wid * b_per_w + t * GCH
            pltpu.sync_copy(idx_hbm.at[pl.ds(base, GCH)], idx_v)
            ca = pltpu.async_copy(fc_hbm.at[idx_v], rows_a, sem_a)
            cb = pltpu.async_copy(fs_hbm.at[idx_v], rows_b, sem_b)
            ca.wait()
            pltpu.sync_copy(rows_a, kg_hbm.at[pl.ds(base, GCH)])
            cb.wait()
            pltpu.sync_copy(rows_b, sg_hbm.at[pl.ds(base, GCH)])
            return carry

        lax.fori_loop(0, n_ch, body, 0)

    return gather_k(fc, fs, idxf)


def kernel(xyz_ref, xyz_pred, feat_coor_ref, feat_coor_pred, feat_sp_ref,
           W_v, b_v, W_o, b_o, W_out, b_out):
    idx = pl.pallas_call(
        _knn_body,
        grid=(NBQ,),
        in_specs=[pl.BlockSpec((B, 3), lambda i: (i, 0)),
                  pl.BlockSpec((3, N_REF), lambda i: (0, 0))],
        out_specs=pl.BlockSpec((B, K), lambda i: (i, 0)),
        out_shape=jax.ShapeDtypeStruct((N_PRED, K), jnp.int32),
    )(xyz_pred, xyz_ref.T)

    kg, sg = _sc_gather(feat_coor_ref, feat_sp_ref, idx.reshape(-1))

    m, c = pl.pallas_call(
        _combine_body,
        out_shape=(jax.ShapeDtypeStruct((C, 2 * C), jnp.float32),
                   jax.ShapeDtypeStruct((1, 2 * C), jnp.float32)),
    )(W_v, W_o, W_out, b_v.reshape(1, C), b_o.reshape(1, C),
      b_out.reshape(1, 2 * C))

    out = pl.pallas_call(
        _attn_body,
        grid=(2 * NBQ,),
        in_specs=[
            pl.BlockSpec((B, C), lambda i: (jnp.maximum(i - NBQ, 0), 0)),
            pl.BlockSpec((B * K, C), lambda i: (jnp.maximum(i - NBQ, 0), 0)),
            pl.BlockSpec((B * K, C), lambda i: (jnp.maximum(i - NBQ, 0), 0)),
            pl.BlockSpec((C, 2 * C), lambda i: (0, 0)),
            pl.BlockSpec((1, 2 * C), lambda i: (0, 0)),
            pl.BlockSpec((1, 2 * C), lambda i: (0, 0)),
        ],
        out_specs=pl.BlockSpec((B, 2 * C), lambda i: (i, 0)),
        out_shape=jax.ShapeDtypeStruct((N_REF + N_PRED, 2 * C), jnp.float32),
    )(feat_coor_pred, kg, sg, m, c, b_out.reshape(1, 2 * C))
    return out


# speedup vs baseline: 2.6344x; 1.0855x over previous
import functools
import math

import jax
import jax.numpy as jnp
from jax import lax
from jax.experimental import pallas as pl
from jax.experimental.pallas import tpu as pltpu
from jax.experimental.pallas import tpu_sc as plsc

N_REF = 8192
N_PRED = 8192
C = 512
K = 8
B = 256
NBQ = N_PRED // B
SCALE = 1.0 / math.sqrt(C)

SC_NC = 2
SC_NS = 16
SC_NW = SC_NC * SC_NS
GCH = 64


def _knn_body(xp_ref, xrt_ref, idx_ref):
    xp = xp_ref[...]
    xpb = xp.astype(jnp.bfloat16).astype(jnp.float32)
    x0 = xrt_ref[0:1, :]
    x1 = xrt_ref[1:2, :]
    x2 = xrt_ref[2:3, :]
    x0b = x0.astype(jnp.bfloat16).astype(jnp.float32)
    x1b = x1.astype(jnp.bfloat16).astype(jnp.float32)
    x2b = x2.astype(jnp.bfloat16).astype(jnp.float32)
    xp2 = jnp.sum(xp * xp, axis=1, keepdims=True)
    xr2 = x0 * x0 + x1 * x1 + x2 * x2
    dot = (xpb[:, 0:1] * x0b + xpb[:, 1:2] * x1b
           + xpb[:, 2:3] * x2b)
    d = xp2 - 2.0 * dot + xr2
    col = lax.broadcasted_iota(jnp.int32, (B, N_REF), 1).astype(jnp.float32)
    picks = []
    for _ in range(K):
        m = jnp.min(d, axis=1, keepdims=True)
        t = jnp.where(d == m, col, jnp.float32(N_REF))
        sel = jnp.min(t, axis=1, keepdims=True)
        picks.append(sel)
        d = jnp.where(t == sel, jnp.inf, d)
    idx_ref[...] = jnp.concatenate(picks, axis=1).astype(jnp.int32)


def _combine_body(wv_ref, wo_ref, wout_ref, bv_ref, bo_ref, bout_ref,
                  m_ref, c_ref):
    a = jnp.dot(wo_ref[...], wv_ref[...],
                preferred_element_type=jnp.float32)
    m_ref[...] = lax.dot_general(a, wout_ref[...], (((0,), (0,)), ((), ())),
                                 preferred_element_type=jnp.float32)
    cb = lax.dot_general(bv_ref[...], wo_ref[...], (((1,), (1,)), ((), ())),
                         preferred_element_type=jnp.float32) + bo_ref[...]
    c_ref[...] = jnp.dot(cb, wout_ref[...],
                         preferred_element_type=jnp.float32) + bout_ref[...]


def _attn_body(q_ref, kg_ref, sg_ref, m_ref, c_ref, bout_ref, out_ref):
    pid = pl.program_id(0)

    @pl.when(pid < NBQ)
    def _():
        out_ref[...] = jnp.broadcast_to(bout_ref[...], (B, 2 * C))

    @pl.when(pid >= NBQ)
    def _():
        q = q_ref[...]
        kg = kg_ref[...].reshape(B, K, C)
        sg = sg_ref[...].reshape(B, K, C)
        s = jnp.sum(q[:, None, :] * kg, axis=2) * SCALE
        mx = jnp.max(s, axis=1, keepdims=True)
        e = jnp.exp(s - mx)
        a = e / jnp.sum(e, axis=1, keepdims=True)
        osum = jnp.sum(a[:, :, None] * sg, axis=1)
        out_ref[...] = jnp.dot(osum, m_ref[...],
                               preferred_element_type=jnp.float32) + c_ref[...]


def _sc_gather(fc, fs, idxf):
    n_idx = idxf.shape[0]
    b_per_w = n_idx // SC_NW
    n_ch = b_per_w // GCH
    mesh = plsc.VectorSubcoreMesh(core_axis_name="c", subcore_axis_name="s")

    @functools.partial(
        pl.kernel, mesh=mesh,
        out_type=[jax.ShapeDtypeStruct((n_idx, C), jnp.float32),
                  jax.ShapeDtypeStruct((n_idx, C), jnp.float32)],
        scratch_types=[pltpu.VMEM((GCH,), jnp.int32),
                       pltpu.VMEM((GCH, C), jnp.float32),
                       pltpu.VMEM((GCH, C), jnp.float32),
                       pltpu.SemaphoreType.DMA,
                       pltpu.SemaphoreType.DMA],
    )
    def gather_k(fc_hbm, fs_hbm, idx_hbm, kg_hbm, sg_hbm,
                 idx_v, rows_a, rows_b, sem_a, sem_b):
        wid = lax.axis_index("s") * SC_NC + lax.axis_index("c")

        def body(t, carry):
            base = wid * b_per_w + t * GCH
            pltpu.sync_copy(idx_hbm.at[pl.ds(base, GCH)], idx_v)
            ca = pltpu.async_copy(fc_hbm.at[idx_v], rows_a, sem_a)
            cb = pltpu.async_copy(fs_hbm.at[idx_v], rows_b, sem_b)
            ca.wait()
            pltpu.sync_copy(rows_a, kg_hbm.at[pl.ds(base, GCH)])
            cb.wait()
            pltpu.sync_copy(rows_b, sg_hbm.at[pl.ds(base, GCH)])
            return carry

        lax.fori_loop(0, n_ch, body, 0)

    return gather_k(fc, fs, idxf)


def kernel(xyz_ref, xyz_pred, feat_coor_ref, feat_coor_pred, feat_sp_ref,
           W_v, b_v, W_o, b_o, W_out, b_out):
    idx = pl.pallas_call(
        _knn_body,
        grid=(NBQ,),
        in_specs=[pl.BlockSpec((B, 3), lambda i: (i, 0)),
                  pl.BlockSpec((3, N_REF), lambda i: (0, 0))],
        out_specs=pl.BlockSpec((B, K), lambda i: (i, 0)),
        out_shape=jax.ShapeDtypeStruct((N_PRED, K), jnp.int32),
    )(xyz_pred, xyz_ref.T)

    kg, sg = _sc_gather(feat_coor_ref, feat_sp_ref, idx.reshape(-1))

    m, c = pl.pallas_call(
        _combine_body,
        out_shape=(jax.ShapeDtypeStruct((C, 2 * C), jnp.float32),
                   jax.ShapeDtypeStruct((1, 2 * C), jnp.float32)),
    )(W_v, W_o, W_out, b_v.reshape(1, C), b_o.reshape(1, C),
      b_out.reshape(1, 2 * C))

    out = pl.pallas_call(
        _attn_body,
        grid=(2 * NBQ,),
        in_specs=[
            pl.BlockSpec((B, C), lambda i: (jnp.maximum(i - NBQ, 0), 0)),
            pl.BlockSpec((B * K, C), lambda i: (jnp.maximum(i - NBQ, 0), 0)),
            pl.BlockSpec((B * K, C), lambda i: (jnp.maximum(i - NBQ, 0), 0)),
            pl.BlockSpec((C, 2 * C), lambda i: (0, 0)),
            pl.BlockSpec((1, 2 * C), lambda i: (0, 0)),
            pl.BlockSpec((1, 2 * C), lambda i: (0, 0)),
        ],
        out_specs=pl.BlockSpec((B, 2 * C), lambda i: (i, 0)),
        out_shape=jax.ShapeDtypeStruct((N_REF + N_PRED, 2 * C), jnp.float32),
    )(feat_coor_pred, kg, sg, m, c, b_out.reshape(1, 2 * C))
    return out


# speedup vs baseline: 2.8216x; 1.0711x over previous
import functools
import math

import jax
import jax.numpy as jnp
from jax import lax
from jax.experimental import pallas as pl
from jax.experimental.pallas import tpu as pltpu
from jax.experimental.pallas import tpu_sc as plsc

N_REF = 8192
N_PRED = 8192
C = 512
K = 8
B = 256
NBQ = N_PRED // B
SCALE = 1.0 / math.sqrt(C)

SC_NC = 2
SC_NS = 16
SC_NW = SC_NC * SC_NS
GCH = 64


def _knn_body(xp2_ref, xr2_ref, xpb_ref, xrb_ref, idx_ref):
    dot = lax.dot_general(xpb_ref[...], xrb_ref[...],
                          (((1,), (1,)), ((), ())),
                          preferred_element_type=jnp.float32)
    d = (xp2_ref[...] - 2.0 * dot) + xr2_ref[...]
    col = lax.broadcasted_iota(jnp.int32, (B, N_REF), 1).astype(jnp.float32)
    picks = []
    for _ in range(K):
        m = jnp.min(d, axis=1, keepdims=True)
        t = jnp.where(d == m, col, jnp.float32(N_REF))
        sel = jnp.min(t, axis=1, keepdims=True)
        picks.append(sel)
        d = jnp.where(t == sel, jnp.inf, d)
    idx_ref[...] = jnp.concatenate(picks, axis=1).astype(jnp.int32)


def _combine_body(wv_ref, wo_ref, wout_ref, bv_ref, bo_ref, bout_ref,
                  m_ref, c_ref):
    a = jnp.dot(wo_ref[...], wv_ref[...],
                preferred_element_type=jnp.float32)
    m_ref[...] = lax.dot_general(a, wout_ref[...], (((0,), (0,)), ((), ())),
                                 preferred_element_type=jnp.float32)
    cb = lax.dot_general(bv_ref[...], wo_ref[...], (((1,), (1,)), ((), ())),
                         preferred_element_type=jnp.float32) + bo_ref[...]
    c_ref[...] = jnp.dot(cb, wout_ref[...],
                         preferred_element_type=jnp.float32) + bout_ref[...]


def _attn_body(q_ref, kg_ref, sg_ref, m_ref, c_ref, bout_ref, out_ref):
    pid = pl.program_id(0)

    @pl.when(pid < NBQ)
    def _():
        out_ref[...] = jnp.broadcast_to(bout_ref[...], (B, 2 * C))

    @pl.when(pid >= NBQ)
    def _():
        q = q_ref[...]
        kg = kg_ref[...].reshape(B, K, C)
        sg = sg_ref[...].reshape(B, K, C)
        s = jnp.sum(q[:, None, :] * kg, axis=2) * SCALE
        mx = jnp.max(s, axis=1, keepdims=True)
        e = jnp.exp(s - mx)
        a = e / jnp.sum(e, axis=1, keepdims=True)
        osum = jnp.sum(a[:, :, None] * sg, axis=1)
        out_ref[...] = jnp.dot(osum, m_ref[...],
                               preferred_element_type=jnp.float32) + c_ref[...]


def _sc_gather(fc, fs, idxf):
    n_idx = idxf.shape[0]
    b_per_w = n_idx // SC_NW
    n_ch = b_per_w // GCH
    mesh = plsc.VectorSubcoreMesh(core_axis_name="c", subcore_axis_name="s")

    @functools.partial(
        pl.kernel, mesh=mesh,
        out_type=[jax.ShapeDtypeStruct((n_idx, C), jnp.float32),
                  jax.ShapeDtypeStruct((n_idx, C), jnp.float32)],
        scratch_types=[pltpu.VMEM((GCH,), jnp.int32),
                       pltpu.VMEM((GCH, C), jnp.float32),
                       pltpu.VMEM((GCH, C), jnp.float32),
                       pltpu.SemaphoreType.DMA,
                       pltpu.SemaphoreType.DMA],
    )
    def gather_k(fc_hbm, fs_hbm, idx_hbm, kg_hbm, sg_hbm,
                 idx_v, rows_a, rows_b, sem_a, sem_b):
        wid = lax.axis_index("s") * SC_NC + lax.axis_index("c")

        def body(t, carry):
            base = wid * b_per_w + t * GCH
            pltpu.sync_copy(idx_hbm.at[pl.ds(base, GCH)], idx_v)
            ca = pltpu.async_copy(fc_hbm.at[idx_v], rows_a, sem_a)
            cb = pltpu.async_copy(fs_hbm.at[idx_v], rows_b, sem_b)
            ca.wait()
            pltpu.sync_copy(rows_a, kg_hbm.at[pl.ds(base, GCH)])
            cb.wait()
            pltpu.sync_copy(rows_b, sg_hbm.at[pl.ds(base, GCH)])
            return carry

        lax.fori_loop(0, n_ch, body, 0)

    return gather_k(fc, fs, idxf)


def kernel(xyz_ref, xyz_pred, feat_coor_ref, feat_coor_pred, feat_sp_ref,
           W_v, b_v, W_o, b_o, W_out, b_out):
    idx = pl.pallas_call(
        _knn_body,
        grid=(NBQ,),
        in_specs=[pl.BlockSpec((B, 1), lambda i: (i, 0)),
                  pl.BlockSpec((1, N_REF), lambda i: (0, 0)),
                  pl.BlockSpec((B, 3), lambda i: (i, 0)),
                  pl.BlockSpec((N_REF, 3), lambda i: (0, 0))],
        out_specs=pl.BlockSpec((B, K), lambda i: (i, 0)),
        out_shape=jax.ShapeDtypeStruct((N_PRED, K), jnp.int32),
    )(jnp.sum(xyz_pred ** 2, axis=1, keepdims=True),
      jnp.sum(xyz_ref ** 2, axis=1)[None, :],
      xyz_pred.astype(jnp.bfloat16), xyz_ref.astype(jnp.bfloat16))

    kg, sg = _sc_gather(feat_coor_ref, feat_sp_ref, idx.reshape(-1))

    m, c = pl.pallas_call(
        _combine_body,
        out_shape=(jax.ShapeDtypeStruct((C, 2 * C), jnp.float32),
                   jax.ShapeDtypeStruct((1, 2 * C), jnp.float32)),
    )(W_v, W_o, W_out, b_v.reshape(1, C), b_o.reshape(1, C),
      b_out.reshape(1, 2 * C))

    out = pl.pallas_call(
        _attn_body,
        grid=(2 * NBQ,),
        in_specs=[
            pl.BlockSpec((B, C), lambda i: (jnp.maximum(i - NBQ, 0), 0)),
            pl.BlockSpec((B * K, C), lambda i: (jnp.maximum(i - NBQ, 0), 0)),
            pl.BlockSpec((B * K, C), lambda i: (jnp.maximum(i - NBQ, 0), 0)),
            pl.BlockSpec((C, 2 * C), lambda i: (0, 0)),
            pl.BlockSpec((1, 2 * C), lambda i: (0, 0)),
            pl.BlockSpec((1, 2 * C), lambda i: (0, 0)),
        ],
        out_specs=pl.BlockSpec((B, 2 * C), lambda i: (i, 0)),
        out_shape=jax.ShapeDtypeStruct((N_REF + N_PRED, 2 * C), jnp.float32),
    )(feat_coor_pred, kg, sg, m, c, b_out.reshape(1, 2 * C))
    return out
